# baseline (speedup 1.0000x reference)
import dataclasses
import functools

import jax
import jax.numpy as jnp
from jax import lax
from jax.experimental import pallas as pl
from jax.experimental.pallas import tpu as pltpu
from jax.experimental.pallas import tpu_sc as plsc

N = 10000
E = 320000
D = 128
DH = D // 2

NC = 2
NS = 16
NW = NC * NS
EPW = E // NW
K = 128
NWF = EPW // K
REM = EPW - NWF * K
NPAD = 10240
RPS = NPAD // NS

_mesh = plsc.VectorSubcoreMesh(core_axis_name="c", subcore_axis_name="s")

_sc_params = dataclasses.replace(
    pltpu.CompilerParams(),
    needs_layout_passes=False,
    use_tc_tiling_on_sc=False,
)


def _tc_pre(x, W_gat, att_src_c, att_dst_c):
    def body(x_ref, w_ref, as_ref, ad_ref,
             h1_ref, h2_ref, a1_ref, a2_ref, c_ref):
        h = jnp.dot(x_ref[...], w_ref[...], preferred_element_type=jnp.float32)
        h1_ref[...] = h[:, :DH]
        h2_ref[...] = h[:, DH:]
        a1 = jnp.dot(h, as_ref[...], preferred_element_type=jnp.float32)
        a2 = jnp.dot(h, ad_ref[...], preferred_element_type=jnp.float32)
        a1_ref[...] = a1
        a2_ref[...] = a2
        m = jnp.max(a1) + jnp.max(a2)
        cc = jnp.where(m >= 0.0, m, 0.2 * m)
        c_ref[...] = jnp.full((16,), cc, jnp.float32)

    return pl.pallas_call(
        body,
        out_shape=(
            jax.ShapeDtypeStruct((N, DH), jnp.float32),
            jax.ShapeDtypeStruct((N, DH), jnp.float32),
            jax.ShapeDtypeStruct((N, 1), jnp.float32),
            jax.ShapeDtypeStruct((N, 1), jnp.float32),
            jax.ShapeDtypeStruct((16,), jnp.float32),
        ),
    )(x, W_gat, att_src_c, att_dst_c)


def _sc_gat(h1, h2, asrc, adst, src, dst, cvec):
    @functools.partial(
        pl.kernel,
        out_type=(
            jax.ShapeDtypeStruct((2, NC, NPAD, DH), jnp.float32),
            jax.ShapeDtypeStruct((NC, NS, NPAD), jnp.float32),
            jax.ShapeDtypeStruct((NC, NS, NPAD), jnp.float32),
        ),
        mesh=_mesh,
        scratch_types=(
            [pltpu.VMEM((N,), jnp.float32)] * 2
            + [pltpu.VMEM((16,), jnp.float32)]
            + [pltpu.VMEM((K,), jnp.int32)] * 4
            + [pltpu.VMEM((K,), jnp.float32)] * 2
            + [pltpu.VMEM((K, DH), jnp.float32)] * 4
            + [pltpu.VMEM((REM,), jnp.int32)] * 2
            + [pltpu.VMEM((REM, DH), jnp.float32)] * 2
            + [pltpu.VMEM((REM,), jnp.float32)]
            + [pltpu.VMEM((NPAD,), jnp.float32)] * 2
            + [pltpu.VMEM_SHARED((NPAD, DH), jnp.float32)]
            + [pltpu.SemaphoreType.DMA] * 4
        ),
        compiler_params=_sc_params,
    )
    def k(h1_hbm, h2_hbm, asrc_hbm, adst_hbm, src_hbm, dst_hbm, c_hbm,
          gat_hbm, den_hbm, deg_hbm,
          asrc_v, adst_v, c_v, src0, src1, dst0, dst1,
          w0, w1, hb0, hb1, rb0, rb1, srcr, dstr, hr, rr, wr,
          den_acc, deg_acc, table,
          semi0, semi1, semg0, semg1):
        c = lax.axis_index("c")
        s = lax.axis_index("s")
        wid = s * NC + c
        pltpu.sync_copy(asrc_hbm, asrc_v)
        pltpu.sync_copy(adst_hbm, adst_v)
        pltpu.sync_copy(c_hbm, c_v)

        SB = (src0, src1)
        DB = (dst0, dst1)
        WB = (w0, w1)
        HB = (hb0, hb1)
        RB = (rb0, rb1)
        SEMI = (semi0, semi1)
        SEMG = (semg0, semg1)

        zeros16 = jnp.zeros((16,), jnp.float32)
        ones16 = jnp.full((16,), 1.0, jnp.float32)

        @pl.loop(0, NPAD // 16)
        def _(i):
            den_acc[pl.ds(i * 16, 16)] = zeros16
            deg_acc[pl.ds(i * 16, 16)] = zeros16

        row0 = s * RPS
        ebase = wid * EPW

        def idx_issue(p, b):
            pltpu.async_copy(src_hbm.at[pl.ds(b, K)], SB[p], SEMI[p])
            pltpu.async_copy(dst_hbm.at[pl.ds(b, K)], DB[p], SEMI[p])

        def idx_wait(p, b):
            pltpu.make_async_copy(src_hbm.at[pl.ds(b, K)], SB[p],
                                  SEMI[p]).wait()
            pltpu.make_async_copy(dst_hbm.at[pl.ds(b, K)], DB[p],
                                  SEMI[p]).wait()

        def edge_w(si, di, accumulate):
            sa = plsc.load_gather(asrc_v, [si])
            da = plsc.load_gather(adst_v, [di])
            t = sa + da
            e = jnp.where(t >= 0.0, t, 0.2 * t)
            w = jnp.exp(e - c_v[...])
            if accumulate:
                plsc.addupdate_scatter(den_acc, [di], w)
                plsc.addupdate_scatter(deg_acc, [di], ones16)
            return w

        for phase, h_hbm in ((0, h1_hbm), (1, h2_hbm)):
            accumulate = phase == 0

            def gat_issue(p, _h=h_hbm):
                pltpu.async_copy(_h.at[SB[p]], HB[p], SEMG[p])

            def gat_wait(p, _h=h_hbm):
                pltpu.make_async_copy(_h.at[SB[p]], HB[p], SEMG[p]).wait()

            def weights(p, _acc=accumulate):
                @pl.loop(0, K // 16)
                def _(j):
                    si = SB[p][pl.ds(j * 16, 16)]
                    di = DB[p][pl.ds(j * 16, 16)]
                    WB[p][pl.ds(j * 16, 16)] = edge_w(si, di, _acc)

            def mult(p):
                @pl.loop(0, K)
                def _(kk):
                    wv = plsc.load_gather(
                        WB[p], [jnp.full((16,), kk, jnp.int32)])
                    for jj in range(DH // 16):
                        RB[p][kk, pl.ds(jj * 16, 16)] = (
                            HB[p][kk, pl.ds(jj * 16, 16)] * wv)

            def scat(p):
                pltpu.sync_copy(RB[p], table.at[DB[p]], add=True)

            @pl.loop(0, K)
            def _(kk):
                for j in range(DH // 16):
                    rb0[kk, pl.ds(j * 16, 16)] = zeros16

            @pl.loop(0, RPS // K)
            def _(i):
                pltpu.sync_copy(rb0, table.at[pl.ds(row0 + i * K, K)])

            plsc.subcore_barrier()

            idx_issue(0, ebase)
            idx_wait(0, ebase)
            gat_issue(0)
            idx_issue(1, ebase + K)

            @pl.loop(0, (NWF - 2) // 2)
            def _(t):
                b0 = ebase + 2 * t * K
                for p in (0, 1):
                    bw = b0 + p * K
                    idx_wait(1 - p, bw + K)
                    gat_issue(1 - p)
                    weights(p)
                    gat_wait(p)
                    mult(p)
                    scat(p)
                    idx_issue(p, bw + 2 * K)

            bw0 = ebase + (NWF - 2) * K
            idx_wait(1, bw0 + K)
            gat_issue(1)
            weights(0)
            gat_wait(0)
            mult(0)
            scat(0)
            weights(1)
            gat_wait(1)
            mult(1)
            scat(1)

            er = ebase + NWF * K
            pltpu.sync_copy(src_hbm.at[pl.ds(er, REM)], srcr)
            pltpu.sync_copy(dst_hbm.at[pl.ds(er, REM)], dstr)
            wr[...] = edge_w(srcr[...], dstr[...], accumulate)
            pltpu.sync_copy(h_hbm.at[srcr], hr)

            @pl.loop(0, REM)
            def _(i):
                wv = plsc.load_gather(wr, [jnp.full((16,), i, jnp.int32)])
                for jj in range(DH // 16):
                    rr[i, pl.ds(jj * 16, 16)] = hr[i, pl.ds(jj * 16, 16)] * wv

            pltpu.sync_copy(rr, table.at[dstr], add=True)

            plsc.subcore_barrier()
            pltpu.sync_copy(table.at[pl.ds(row0, RPS)],
                            gat_hbm.at[phase, c, pl.ds(row0, RPS)])

        pltpu.sync_copy(den_acc, den_hbm.at[c, s])
        pltpu.sync_copy(deg_acc, deg_hbm.at[c, s])

    return k(h1, h2, asrc, adst, src, dst, cvec)


def _tc_mid(gat_p, den_p, deg_p, b_gat):
    def body(g_ref, dn_ref, dg_ref, b_ref, og1_ref, og2_ref, deg_ref):
        ga = g_ref[0, 0] + g_ref[0, 1]
        gb = g_ref[1, 0] + g_ref[1, 1]
        den = jnp.sum(dn_ref[...], axis=0).reshape(NPAD, 1)
        deg = jnp.sum(dg_ref[...], axis=0).reshape(NPAD, 1)
        recip = 1.0 / jnp.maximum(den[:N], 1e-16)
        b = b_ref[...]
        og1_ref[...] = ga[:N] * recip + b[:DH][None, :]
        og2_ref[...] = gb[:N] * recip + b[DH:][None, :]
        deg_ref[...] = deg[:N]

    return pl.pallas_call(
        body,
        out_shape=(
            jax.ShapeDtypeStruct((N, DH), jnp.float32),
            jax.ShapeDtypeStruct((N, DH), jnp.float32),
            jax.ShapeDtypeStruct((N, 1), jnp.float32),
        ),
    )(gat_p, den_p, deg_p, b_gat)


def _sc_agg(og1, og2, src, dst):
    @functools.partial(
        pl.kernel,
        out_type=jax.ShapeDtypeStruct((2, NC, NPAD, DH), jnp.float32),
        mesh=_mesh,
        scratch_types=(
            [pltpu.VMEM((K,), jnp.int32)] * 6
            + [pltpu.VMEM((K, DH), jnp.float32)] * 3
            + [pltpu.VMEM((REM,), jnp.int32)] * 2
            + [pltpu.VMEM((REM, DH), jnp.float32)]
            + [pltpu.VMEM_SHARED((NPAD, DH), jnp.float32)]
            + [pltpu.SemaphoreType.DMA] * 5
        ),
        compiler_params=_sc_params,
    )
    def k(g1_hbm, g2_hbm, src_hbm, dst_hbm, out_hbm,
          src0, src1, src2, dst0, dst1, dst2, hb0, hb1, hb2,
          srcr, dstr, hr, table,
          semi, semg0, semg1, semg2, sems):
        c = lax.axis_index("c")
        s = lax.axis_index("s")
        wid = s * NC + c
        row0 = s * RPS
        ebase = wid * EPW

        SB = (src0, src1, src2)
        DB = (dst0, dst1, dst2)
        HB = (hb0, hb1, hb2)
        SEMG = (semg0, semg1, semg2)

        zeros16 = jnp.zeros((16,), jnp.float32)

        def idx_issue(p, b):
            pltpu.async_copy(src_hbm.at[pl.ds(b, K)], SB[p], semi)
            pltpu.async_copy(dst_hbm.at[pl.ds(b, K)], DB[p], semi)

        def idx_wait(p, b):
            pltpu.make_async_copy(src_hbm.at[pl.ds(b, K)], SB[p],
                                  semi).wait()
            pltpu.make_async_copy(dst_hbm.at[pl.ds(b, K)], DB[p],
                                  semi).wait()

        for phase, g_hbm in ((0, g1_hbm), (1, g2_hbm)):
            def gat_issue(p, _g=g_hbm):
                pltpu.async_copy(_g.at[SB[p]], HB[p], SEMG[p])

            def gat_wait(p, _g=g_hbm):
                pltpu.make_async_copy(_g.at[SB[p]], HB[p], SEMG[p]).wait()

            def scat_issue(p):
                pltpu.async_copy(HB[p], table.at[DB[p]], sems, add=True)

            def scat_wait(p):
                pltpu.make_async_copy(HB[p], table.at[DB[p]],
                                      sems).wait()

            def body(bw, p, prev_scat=True, nxt=True, nxt2=True):
                q = (p + 1) % 3
                r = (p + 2) % 3
                if nxt:
                    idx_wait(q, bw + K)
                    gat_issue(q)
                if prev_scat:
                    scat_wait(r)
                if nxt2:
                    idx_issue(r, bw + 2 * K)
                gat_wait(p)
                scat_issue(p)

            @pl.loop(0, K)
            def _(kk):
                for j in range(DH // 16):
                    hb0[kk, pl.ds(j * 16, 16)] = zeros16

            @pl.loop(0, RPS // K)
            def _(i):
                pltpu.sync_copy(hb0, table.at[pl.ds(row0 + i * K, K)])

            plsc.subcore_barrier()

            idx_issue(0, ebase)
            idx_wait(0, ebase)
            gat_issue(0)
            idx_issue(1, ebase + K)
            body(ebase, 0, prev_scat=False)

            @pl.loop(0, (NWF - 3) // 3)
            def _(t):
                b0 = ebase + (3 * t + 1) * K
                body(b0, 1)
                body(b0 + K, 2)
                body(b0 + 2 * K, 0)

            bw0 = ebase + (NWF - 2) * K
            body(bw0, 1, nxt2=False)
            body(bw0 + K, 2, nxt=False, nxt2=False)
            scat_wait(2)

            er = ebase + NWF * K
            pltpu.sync_copy(src_hbm.at[pl.ds(er, REM)], srcr)
            pltpu.sync_copy(dst_hbm.at[pl.ds(er, REM)], dstr)
            pltpu.sync_copy(g_hbm.at[srcr], hr)
            pltpu.sync_copy(hr, table.at[dstr], add=True)

            plsc.subcore_barrier()
            pltpu.sync_copy(table.at[pl.ds(row0, RPS)],
                            out_hbm.at[phase, c, pl.ds(row0, RPS)])

    return k(og1, og2, src, dst)


def _tc_post(agg_p, og1, og2, deg, W_l, b_l, W_r):
    def body(a_ref, g1_ref, g2_ref, d_ref, wl_ref, bl_ref, wr_ref, o_ref):
        a1 = a_ref[0, 0] + a_ref[0, 1]
        a2 = a_ref[1, 0] + a_ref[1, 1]
        a = (jnp.concatenate([a1[:N], a2[:N]], axis=1)
             / jnp.maximum(d_ref[...], 1.0))
        og = jnp.concatenate([g1_ref[...], g2_ref[...]], axis=1)
        out = (jnp.dot(a, wl_ref[...], preferred_element_type=jnp.float32)
               + jnp.dot(og, wr_ref[...],
                         preferred_element_type=jnp.float32)
               + bl_ref[...][None, :])
        nrm = jnp.sqrt(jnp.sum(out * out, axis=1, keepdims=True))
        o_ref[...] = out / jnp.maximum(nrm, 1e-12)

    return pl.pallas_call(
        body,
        out_shape=jax.ShapeDtypeStruct((N, D), jnp.float32),
    )(agg_p, og1, og2, deg, W_l, b_l, W_r)


def kernel(x, edge_index, W_gat, att_src, att_dst, b_gat, W_l, b_l, W_r):
    src = edge_index[0]
    dst = edge_index[1]
    h1, h2, asrc2, adst2, cvec = _tc_pre(
        x, W_gat, att_src.reshape(D, 1), att_dst.reshape(D, 1))
    asrc = asrc2.reshape(N)
    adst = adst2.reshape(N)
    gat_p, den_p, deg_p = _sc_gat(h1, h2, asrc, adst, src, dst, cvec)
    og1, og2, deg = _tc_mid(gat_p, den_p.reshape(NW, NPAD),
                            deg_p.reshape(NW, NPAD), b_gat)
    agg_p = _sc_agg(og1, og2, src, dst)
    return _tc_post(agg_p, og1, og2, deg, W_l, b_l, W_r)


# speedup vs baseline: 1.6719x; 1.6719x over previous
import dataclasses
import functools

import jax
import jax.numpy as jnp
from jax import lax
from jax.experimental import pallas as pl
from jax.experimental.pallas import tpu as pltpu
from jax.experimental.pallas import tpu_sc as plsc

N = 10000
E = 320000
D = 128
DH = D // 2

NC = 2
NS = 16
NW = NC * NS
EPW = E // NW
K = 128
NWF = EPW // K
REM = EPW - NWF * K
NPAD = 10240
RPS = NPAD // NS

_mesh = plsc.VectorSubcoreMesh(core_axis_name="c", subcore_axis_name="s")

_sc_params = dataclasses.replace(
    pltpu.CompilerParams(),
    needs_layout_passes=False,
    use_tc_tiling_on_sc=False,
)


def _tc_pre(x, W_gat, att_src_c, att_dst_c):
    def body(x_ref, w_ref, as_ref, ad_ref,
             h1_ref, h2_ref, a1_ref, a2_ref, c_ref):
        h = jnp.dot(x_ref[...], w_ref[...], preferred_element_type=jnp.float32)
        h1_ref[...] = h[:, :DH]
        h2_ref[...] = h[:, DH:]
        a1 = jnp.dot(h, as_ref[...], preferred_element_type=jnp.float32)
        a2 = jnp.dot(h, ad_ref[...], preferred_element_type=jnp.float32)
        a1_ref[...] = a1
        a2_ref[...] = a2
        m = jnp.max(a1) + jnp.max(a2)
        cc = jnp.where(m >= 0.0, m, 0.2 * m)
        c_ref[...] = jnp.full((16,), cc, jnp.float32)

    return pl.pallas_call(
        body,
        out_shape=(
            jax.ShapeDtypeStruct((N, DH), jnp.float32),
            jax.ShapeDtypeStruct((N, DH), jnp.float32),
            jax.ShapeDtypeStruct((N, 1), jnp.float32),
            jax.ShapeDtypeStruct((N, 1), jnp.float32),
            jax.ShapeDtypeStruct((16,), jnp.float32),
        ),
    )(x, W_gat, att_src_c, att_dst_c)


def _sc_gat(h1, h2, asrc, adst, src, dst, cvec):
    @functools.partial(
        pl.kernel,
        out_type=(
            jax.ShapeDtypeStruct((2, NC, NPAD, DH), jnp.float32),
            jax.ShapeDtypeStruct((NC, NS, NPAD), jnp.float32),
            jax.ShapeDtypeStruct((NC, NS, NPAD), jnp.float32),
        ),
        mesh=_mesh,
        scratch_types=(
            [pltpu.VMEM((N,), jnp.float32)] * 2
            + [pltpu.VMEM((16,), jnp.float32)]
            + [pltpu.VMEM((K,), jnp.int32)] * 4
            + [pltpu.VMEM((K,), jnp.float32)] * 2
            + [pltpu.VMEM((K, DH), jnp.float32)] * 4
            + [pltpu.VMEM((REM,), jnp.int32)] * 2
            + [pltpu.VMEM((REM, DH), jnp.float32)] * 2
            + [pltpu.VMEM((REM,), jnp.float32)]
            + [pltpu.VMEM((NPAD,), jnp.float32)] * 2
            + [pltpu.VMEM_SHARED((NPAD, DH), jnp.float32)]
            + [pltpu.SemaphoreType.DMA] * 4
        ),
        compiler_params=_sc_params,
    )
    def k(h1_hbm, h2_hbm, asrc_hbm, adst_hbm, src_hbm, dst_hbm, c_hbm,
          gat_hbm, den_hbm, deg_hbm,
          asrc_v, adst_v, c_v, src0, src1, dst0, dst1,
          w0, w1, hb0, hb1, rb0, rb1, srcr, dstr, hr, rr, wr,
          den_acc, deg_acc, table,
          semi0, semi1, semg0, semg1):
        c = lax.axis_index("c")
        s = lax.axis_index("s")
        wid = s * NC + c
        pltpu.sync_copy(asrc_hbm, asrc_v)
        pltpu.sync_copy(adst_hbm, adst_v)
        pltpu.sync_copy(c_hbm, c_v)

        SB = (src0, src1)
        DB = (dst0, dst1)
        WB = (w0, w1)
        HB = (hb0, hb1)
        RB = (rb0, rb1)
        SEMI = (semi0, semi1)
        SEMG = (semg0, semg1)

        zeros16 = jnp.zeros((16,), jnp.float32)
        ones16 = jnp.full((16,), 1.0, jnp.float32)

        @pl.loop(0, NPAD // 16)
        def _(i):
            den_acc[pl.ds(i * 16, 16)] = zeros16
            deg_acc[pl.ds(i * 16, 16)] = zeros16

        row0 = s * RPS
        ebase = wid * EPW

        def idx_issue(p, b):
            pltpu.async_copy(src_hbm.at[pl.ds(b, K)], SB[p], SEMI[p])
            pltpu.async_copy(dst_hbm.at[pl.ds(b, K)], DB[p], SEMI[p])

        def idx_wait(p, b):
            pltpu.make_async_copy(src_hbm.at[pl.ds(b, K)], SB[p],
                                  SEMI[p]).wait()
            pltpu.make_async_copy(dst_hbm.at[pl.ds(b, K)], DB[p],
                                  SEMI[p]).wait()

        def edge_w(si, di, accumulate):
            sa = plsc.load_gather(asrc_v, [si])
            da = plsc.load_gather(adst_v, [di])
            t = sa + da
            e = jnp.where(t >= 0.0, t, 0.2 * t)
            w = jnp.exp(e - c_v[...])
            if accumulate:
                plsc.addupdate_scatter(den_acc, [di], w)
                plsc.addupdate_scatter(deg_acc, [di], ones16)
            return w

        for phase, h_hbm in ((0, h1_hbm), (1, h2_hbm)):
            accumulate = phase == 0

            def gat_issue(p, _h=h_hbm):
                pltpu.async_copy(_h.at[SB[p]], HB[p], SEMG[p])

            def gat_wait(p, _h=h_hbm):
                pltpu.make_async_copy(_h.at[SB[p]], HB[p], SEMG[p]).wait()

            def weights(p, _acc=accumulate):
                @pl.loop(0, K // 16)
                def _(j):
                    si = SB[p][pl.ds(j * 16, 16)]
                    di = DB[p][pl.ds(j * 16, 16)]
                    WB[p][pl.ds(j * 16, 16)] = edge_w(si, di, _acc)

            def mult(p):
                @pl.loop(0, K // 16)
                def _(j):
                    w16 = WB[p][pl.ds(j * 16, 16)]
                    for i in range(16):
                        kk = j * 16 + i
                        wv = jnp.full((16,), w16[i], jnp.float32)
                        for jj in range(DH // 16):
                            RB[p][kk, pl.ds(jj * 16, 16)] = (
                                HB[p][kk, pl.ds(jj * 16, 16)] * wv)

            def scat(p):
                pltpu.sync_copy(RB[p], table.at[DB[p]], add=True)

            @pl.loop(0, K)
            def _(kk):
                for j in range(DH // 16):
                    rb0[kk, pl.ds(j * 16, 16)] = zeros16

            @pl.loop(0, RPS // K)
            def _(i):
                pltpu.sync_copy(rb0, table.at[pl.ds(row0 + i * K, K)])

            plsc.subcore_barrier()

            idx_issue(0, ebase)
            idx_wait(0, ebase)
            gat_issue(0)
            idx_issue(1, ebase + K)

            @pl.loop(0, (NWF - 2) // 2)
            def _(t):
                b0 = ebase + 2 * t * K
                for p in (0, 1):
                    bw = b0 + p * K
                    idx_wait(1 - p, bw + K)
                    gat_issue(1 - p)
                    weights(p)
                    gat_wait(p)
                    mult(p)
                    scat(p)
                    idx_issue(p, bw + 2 * K)

            bw0 = ebase + (NWF - 2) * K
            idx_wait(1, bw0 + K)
            gat_issue(1)
            weights(0)
            gat_wait(0)
            mult(0)
            scat(0)
            weights(1)
            gat_wait(1)
            mult(1)
            scat(1)

            er = ebase + NWF * K
            pltpu.sync_copy(src_hbm.at[pl.ds(er, REM)], srcr)
            pltpu.sync_copy(dst_hbm.at[pl.ds(er, REM)], dstr)
            w16 = edge_w(srcr[...], dstr[...], accumulate)
            pltpu.sync_copy(h_hbm.at[srcr], hr)
            for i in range(REM):
                wv = jnp.full((16,), w16[i], jnp.float32)
                for jj in range(DH // 16):
                    rr[i, pl.ds(jj * 16, 16)] = hr[i, pl.ds(jj * 16, 16)] * wv
            pltpu.sync_copy(rr, table.at[dstr], add=True)

            plsc.subcore_barrier()
            pltpu.sync_copy(table.at[pl.ds(row0, RPS)],
                            gat_hbm.at[phase, c, pl.ds(row0, RPS)])

        pltpu.sync_copy(den_acc, den_hbm.at[c, s])
        pltpu.sync_copy(deg_acc, deg_hbm.at[c, s])

    return k(h1, h2, asrc, adst, src, dst, cvec)


def _tc_mid(gat_p, den_p, deg_p, b_gat):
    def body(g_ref, dn_ref, dg_ref, b_ref, og1_ref, og2_ref, deg_ref):
        ga = g_ref[0, 0] + g_ref[0, 1]
        gb = g_ref[1, 0] + g_ref[1, 1]
        den = jnp.sum(dn_ref[...], axis=0).reshape(NPAD, 1)
        deg = jnp.sum(dg_ref[...], axis=0).reshape(NPAD, 1)
        recip = 1.0 / jnp.maximum(den[:N], 1e-16)
        b = b_ref[...]
        og1_ref[...] = ga[:N] * recip + b[:DH][None, :]
        og2_ref[...] = gb[:N] * recip + b[DH:][None, :]
        deg_ref[...] = deg[:N]

    return pl.pallas_call(
        body,
        out_shape=(
            jax.ShapeDtypeStruct((N, DH), jnp.float32),
            jax.ShapeDtypeStruct((N, DH), jnp.float32),
            jax.ShapeDtypeStruct((N, 1), jnp.float32),
        ),
    )(gat_p, den_p, deg_p, b_gat)


def _sc_agg(og1, og2, src, dst):
    @functools.partial(
        pl.kernel,
        out_type=jax.ShapeDtypeStruct((2, NC, NPAD, DH), jnp.float32),
        mesh=_mesh,
        scratch_types=(
            [pltpu.VMEM((K,), jnp.int32)] * 6
            + [pltpu.VMEM((K, DH), jnp.float32)] * 3
            + [pltpu.VMEM((REM,), jnp.int32)] * 2
            + [pltpu.VMEM((REM, DH), jnp.float32)]
            + [pltpu.VMEM_SHARED((NPAD, DH), jnp.float32)]
            + [pltpu.SemaphoreType.DMA] * 5
        ),
        compiler_params=_sc_params,
    )
    def k(g1_hbm, g2_hbm, src_hbm, dst_hbm, out_hbm,
          src0, src1, src2, dst0, dst1, dst2, hb0, hb1, hb2,
          srcr, dstr, hr, table,
          semi, semg0, semg1, semg2, sems):
        c = lax.axis_index("c")
        s = lax.axis_index("s")
        wid = s * NC + c
        row0 = s * RPS
        ebase = wid * EPW

        SB = (src0, src1, src2)
        DB = (dst0, dst1, dst2)
        HB = (hb0, hb1, hb2)
        SEMG = (semg0, semg1, semg2)

        zeros16 = jnp.zeros((16,), jnp.float32)

        def idx_issue(p, b):
            pltpu.async_copy(src_hbm.at[pl.ds(b, K)], SB[p], semi)
            pltpu.async_copy(dst_hbm.at[pl.ds(b, K)], DB[p], semi)

        def idx_wait(p, b):
            pltpu.make_async_copy(src_hbm.at[pl.ds(b, K)], SB[p],
                                  semi).wait()
            pltpu.make_async_copy(dst_hbm.at[pl.ds(b, K)], DB[p],
                                  semi).wait()

        for phase, g_hbm in ((0, g1_hbm), (1, g2_hbm)):
            def gat_issue(p, _g=g_hbm):
                pltpu.async_copy(_g.at[SB[p]], HB[p], SEMG[p])

            def gat_wait(p, _g=g_hbm):
                pltpu.make_async_copy(_g.at[SB[p]], HB[p], SEMG[p]).wait()

            def scat_issue(p):
                pltpu.async_copy(HB[p], table.at[DB[p]], sems, add=True)

            def scat_wait(p):
                pltpu.make_async_copy(HB[p], table.at[DB[p]],
                                      sems).wait()

            def body(bw, p, prev_scat=True, nxt=True, nxt2=True):
                q = (p + 1) % 3
                r = (p + 2) % 3
                if nxt:
                    idx_wait(q, bw + K)
                    gat_issue(q)
                if prev_scat:
                    scat_wait(r)
                if nxt2:
                    idx_issue(r, bw + 2 * K)
                gat_wait(p)
                scat_issue(p)

            @pl.loop(0, K)
            def _(kk):
                for j in range(DH // 16):
                    hb0[kk, pl.ds(j * 16, 16)] = zeros16

            @pl.loop(0, RPS // K)
            def _(i):
                pltpu.sync_copy(hb0, table.at[pl.ds(row0 + i * K, K)])

            plsc.subcore_barrier()

            idx_issue(0, ebase)
            idx_wait(0, ebase)
            gat_issue(0)
            idx_issue(1, ebase + K)
            body(ebase, 0, prev_scat=False)

            @pl.loop(0, (NWF - 3) // 3)
            def _(t):
                b0 = ebase + (3 * t + 1) * K
                body(b0, 1)
                body(b0 + K, 2)
                body(b0 + 2 * K, 0)

            bw0 = ebase + (NWF - 2) * K
            body(bw0, 1, nxt2=False)
            body(bw0 + K, 2, nxt=False, nxt2=False)
            scat_wait(2)

            er = ebase + NWF * K
            pltpu.sync_copy(src_hbm.at[pl.ds(er, REM)], srcr)
            pltpu.sync_copy(dst_hbm.at[pl.ds(er, REM)], dstr)
            pltpu.sync_copy(g_hbm.at[srcr], hr)
            pltpu.sync_copy(hr, table.at[dstr], add=True)

            plsc.subcore_barrier()
            pltpu.sync_copy(table.at[pl.ds(row0, RPS)],
                            out_hbm.at[phase, c, pl.ds(row0, RPS)])

    return k(og1, og2, src, dst)


def _tc_post(agg_p, og1, og2, deg, W_l, b_l, W_r):
    def body(a_ref, g1_ref, g2_ref, d_ref, wl_ref, bl_ref, wr_ref, o_ref):
        a1 = a_ref[0, 0] + a_ref[0, 1]
        a2 = a_ref[1, 0] + a_ref[1, 1]
        a = (jnp.concatenate([a1[:N], a2[:N]], axis=1)
             / jnp.maximum(d_ref[...], 1.0))
        og = jnp.concatenate([g1_ref[...], g2_ref[...]], axis=1)
        out = (jnp.dot(a, wl_ref[...], preferred_element_type=jnp.float32)
               + jnp.dot(og, wr_ref[...],
                         preferred_element_type=jnp.float32)
               + bl_ref[...][None, :])
        nrm = jnp.sqrt(jnp.sum(out * out, axis=1, keepdims=True))
        o_ref[...] = out / jnp.maximum(nrm, 1e-12)

    return pl.pallas_call(
        body,
        out_shape=jax.ShapeDtypeStruct((N, D), jnp.float32),
    )(agg_p, og1, og2, deg, W_l, b_l, W_r)


def kernel(x, edge_index, W_gat, att_src, att_dst, b_gat, W_l, b_l, W_r):
    src = edge_index[0]
    dst = edge_index[1]
    h1, h2, asrc2, adst2, cvec = _tc_pre(
        x, W_gat, att_src.reshape(D, 1), att_dst.reshape(D, 1))
    asrc = asrc2.reshape(N)
    adst = adst2.reshape(N)
    gat_p, den_p, deg_p = _sc_gat(h1, h2, asrc, adst, src, dst, cvec)
    og1, og2, deg = _tc_mid(gat_p, den_p.reshape(NW, NPAD),
                            deg_p.reshape(NW, NPAD), b_gat)
    agg_p = _sc_agg(og1, og2, src, dst)
    return _tc_post(agg_p, og1, og2, deg, W_l, b_l, W_r)


# speedup vs baseline: 1.8552x; 1.1096x over previous
import dataclasses
import functools

import jax
import jax.numpy as jnp
from jax import lax
from jax.experimental import pallas as pl
from jax.experimental.pallas import tpu as pltpu
from jax.experimental.pallas import tpu_sc as plsc

N = 10000
E = 320000
D = 128
DH = D // 2

NC = 2
NS = 16
NW = NC * NS
EPW = E // NW
K = 128
NWF = EPW // K
REM = EPW - NWF * K
NPAD = 10240
RPS = NPAD // NS

_mesh = plsc.VectorSubcoreMesh(core_axis_name="c", subcore_axis_name="s")

_sc_params = dataclasses.replace(
    pltpu.CompilerParams(),
    needs_layout_passes=False,
    use_tc_tiling_on_sc=False,
)


def _tc_pre(x, W_gat, att_src_c, att_dst_c):
    def body(x_ref, w_ref, as_ref, ad_ref,
             h1_ref, h2_ref, a1_ref, a2_ref, c_ref):
        h = jnp.dot(x_ref[...], w_ref[...], preferred_element_type=jnp.float32)
        h1_ref[...] = h[:, :DH]
        h2_ref[...] = h[:, DH:]
        a1 = jnp.dot(h, as_ref[...], preferred_element_type=jnp.float32)
        a2 = jnp.dot(h, ad_ref[...], preferred_element_type=jnp.float32)
        a1_ref[...] = a1
        a2_ref[...] = a2
        m = jnp.max(a1) + jnp.max(a2)
        cc = jnp.where(m >= 0.0, m, 0.2 * m)
        c_ref[...] = jnp.full((16,), cc, jnp.float32)

    return pl.pallas_call(
        body,
        out_shape=(
            jax.ShapeDtypeStruct((N, DH), jnp.float32),
            jax.ShapeDtypeStruct((N, DH), jnp.float32),
            jax.ShapeDtypeStruct((N, 1), jnp.float32),
            jax.ShapeDtypeStruct((N, 1), jnp.float32),
            jax.ShapeDtypeStruct((16,), jnp.float32),
        ),
    )(x, W_gat, att_src_c, att_dst_c)


def _sc_gat(h1, h2, asrc, adst, src, dst, cvec):
    @functools.partial(
        pl.kernel,
        out_type=(
            jax.ShapeDtypeStruct((2, NC, NPAD, DH), jnp.float32),
            jax.ShapeDtypeStruct((NC, NS, NPAD), jnp.float32),
            jax.ShapeDtypeStruct((NC, NS, NPAD), jnp.float32),
        ),
        mesh=_mesh,
        scratch_types=(
            [pltpu.VMEM((N,), jnp.float32)] * 2
            + [pltpu.VMEM((16,), jnp.float32)]
            + [pltpu.VMEM((K,), jnp.int32)] * 6
            + [pltpu.VMEM((K,), jnp.float32)] * 2
            + [pltpu.VMEM((K, DH), jnp.float32)] * 4
            + [pltpu.VMEM((REM,), jnp.int32)] * 2
            + [pltpu.VMEM((REM, DH), jnp.float32)] * 2
            + [pltpu.VMEM((REM,), jnp.float32)]
            + [pltpu.VMEM((NPAD,), jnp.float32)] * 2
            + [pltpu.VMEM_SHARED((NPAD, DH), jnp.float32)]
            + [pltpu.SemaphoreType.DMA] * 6
        ),
        compiler_params=_sc_params,
    )
    def k(h1_hbm, h2_hbm, asrc_hbm, adst_hbm, src_hbm, dst_hbm, c_hbm,
          gat_hbm, den_hbm, deg_hbm,
          asrc_v, adst_v, c_v, src0, src1, dst0, dst1, sdst0, sdst1,
          w0, w1, hb0, hb1, rb0, rb1, srcr, dstr, hr, rr, wr,
          den_acc, deg_acc, table,
          semi0, semi1, semg0, semg1, sems0, sems1):
        c = lax.axis_index("c")
        s = lax.axis_index("s")
        wid = s * NC + c
        pltpu.sync_copy(asrc_hbm, asrc_v)
        pltpu.sync_copy(adst_hbm, adst_v)
        pltpu.sync_copy(c_hbm, c_v)

        SB = (src0, src1)
        DB = (dst0, dst1)
        SD = (sdst0, sdst1)
        WB = (w0, w1)
        HB = (hb0, hb1)
        RB = (rb0, rb1)
        SEMI = (semi0, semi1)
        SEMG = (semg0, semg1)
        SEMS = (sems0, sems1)

        zeros16 = jnp.zeros((16,), jnp.float32)
        ones16 = jnp.full((16,), 1.0, jnp.float32)

        @pl.loop(0, NPAD // 16)
        def _(i):
            den_acc[pl.ds(i * 16, 16)] = zeros16
            deg_acc[pl.ds(i * 16, 16)] = zeros16

        row0 = s * RPS
        ebase = wid * EPW

        def idx_issue(p, b):
            pltpu.async_copy(src_hbm.at[pl.ds(b, K)], SB[p], SEMI[p])
            pltpu.async_copy(dst_hbm.at[pl.ds(b, K)], DB[p], SEMI[p])

        def idx_wait(p, b):
            pltpu.make_async_copy(src_hbm.at[pl.ds(b, K)], SB[p],
                                  SEMI[p]).wait()
            pltpu.make_async_copy(dst_hbm.at[pl.ds(b, K)], DB[p],
                                  SEMI[p]).wait()

        def edge_w(si, di, accumulate):
            sa = plsc.load_gather(asrc_v, [si])
            da = plsc.load_gather(adst_v, [di])
            t = sa + da
            e = jnp.where(t >= 0.0, t, 0.2 * t)
            w = jnp.exp(e - c_v[...])
            if accumulate:
                plsc.addupdate_scatter(den_acc, [di], w)
                plsc.addupdate_scatter(deg_acc, [di], ones16)
            return w

        for phase, h_hbm in ((0, h1_hbm), (1, h2_hbm)):
            accumulate = phase == 0

            def gat_issue(p, _h=h_hbm):
                pltpu.async_copy(_h.at[SB[p]], HB[p], SEMG[p])

            def gat_wait(p, _h=h_hbm):
                pltpu.make_async_copy(_h.at[SB[p]], HB[p], SEMG[p]).wait()

            def weights(p, _acc=accumulate):
                @pl.loop(0, K // 16)
                def _(j):
                    si = SB[p][pl.ds(j * 16, 16)]
                    di = DB[p][pl.ds(j * 16, 16)]
                    WB[p][pl.ds(j * 16, 16)] = edge_w(si, di, _acc)

            def mult(p):
                @pl.loop(0, K // 16)
                def _(j):
                    w16 = WB[p][pl.ds(j * 16, 16)]
                    for i in range(16):
                        kk = j * 16 + i
                        wv = jnp.full((16,), w16[i], jnp.float32)
                        for jj in range(DH // 16):
                            RB[p][kk, pl.ds(jj * 16, 16)] = (
                                HB[p][kk, pl.ds(jj * 16, 16)] * wv)

            def scat_issue(p):
                pltpu.async_copy(RB[p], table.at[SD[p]], SEMS[p], add=True)

            def scat_wait(p):
                pltpu.make_async_copy(RB[p], table.at[SD[p]],
                                      SEMS[p]).wait()

            @pl.loop(0, K)
            def _(kk):
                for j in range(DH // 16):
                    rb0[kk, pl.ds(j * 16, 16)] = zeros16

            @pl.loop(0, RPS // K)
            def _(i):
                pltpu.sync_copy(rb0, table.at[pl.ds(row0 + i * K, K)])

            plsc.subcore_barrier()

            idx_issue(0, ebase)
            idx_wait(0, ebase)
            gat_issue(0)
            idx_issue(1, ebase + K)

            def sweep_body(bw, p, first_pair=False):
                idx_wait(1 - p, bw + K)
                gat_issue(1 - p)
                weights(p)
                if not first_pair:
                    scat_wait(p)
                for j in range(K // 16):
                    SD[p][pl.ds(j * 16, 16)] = DB[p][pl.ds(j * 16, 16)]
                gat_wait(p)
                mult(p)
                scat_issue(p)
                idx_issue(p, bw + 2 * K)

            sweep_body(ebase, 0, first_pair=True)
            sweep_body(ebase + K, 1, first_pair=True)

            @pl.loop(0, (NWF - 4) // 2)
            def _(t):
                b0 = ebase + (2 * t + 2) * K
                sweep_body(b0, 0)
                sweep_body(b0 + K, 1)

            bw0 = ebase + (NWF - 2) * K
            idx_wait(1, bw0 + K)
            gat_issue(1)
            weights(0)
            scat_wait(0)
            for j in range(K // 16):
                SD[0][pl.ds(j * 16, 16)] = DB[0][pl.ds(j * 16, 16)]
            gat_wait(0)
            mult(0)
            scat_issue(0)
            weights(1)
            scat_wait(1)
            for j in range(K // 16):
                SD[1][pl.ds(j * 16, 16)] = DB[1][pl.ds(j * 16, 16)]
            gat_wait(1)
            mult(1)
            scat_issue(1)
            scat_wait(0)
            scat_wait(1)

            er = ebase + NWF * K
            pltpu.sync_copy(src_hbm.at[pl.ds(er, REM)], srcr)
            pltpu.sync_copy(dst_hbm.at[pl.ds(er, REM)], dstr)
            w16 = edge_w(srcr[...], dstr[...], accumulate)
            pltpu.sync_copy(h_hbm.at[srcr], hr)
            for i in range(REM):
                wv = jnp.full((16,), w16[i], jnp.float32)
                for jj in range(DH // 16):
                    rr[i, pl.ds(jj * 16, 16)] = hr[i, pl.ds(jj * 16, 16)] * wv
            pltpu.sync_copy(rr, table.at[dstr], add=True)

            plsc.subcore_barrier()
            pltpu.sync_copy(table.at[pl.ds(row0, RPS)],
                            gat_hbm.at[phase, c, pl.ds(row0, RPS)])

        pltpu.sync_copy(den_acc, den_hbm.at[c, s])
        pltpu.sync_copy(deg_acc, deg_hbm.at[c, s])

    return k(h1, h2, asrc, adst, src, dst, cvec)


def _tc_mid(gat_p, den_p, deg_p, b_gat):
    def body(g_ref, dn_ref, dg_ref, b_ref, og1_ref, og2_ref, deg_ref):
        ga = g_ref[0, 0] + g_ref[0, 1]
        gb = g_ref[1, 0] + g_ref[1, 1]
        den = jnp.sum(dn_ref[...], axis=0).reshape(NPAD, 1)
        deg = jnp.sum(dg_ref[...], axis=0).reshape(NPAD, 1)
        recip = 1.0 / jnp.maximum(den[:N], 1e-16)
        b = b_ref[...]
        og1_ref[...] = ga[:N] * recip + b[:DH][None, :]
        og2_ref[...] = gb[:N] * recip + b[DH:][None, :]
        deg_ref[...] = deg[:N]

    return pl.pallas_call(
        body,
        out_shape=(
            jax.ShapeDtypeStruct((N, DH), jnp.float32),
            jax.ShapeDtypeStruct((N, DH), jnp.float32),
            jax.ShapeDtypeStruct((N, 1), jnp.float32),
        ),
    )(gat_p, den_p, deg_p, b_gat)


def _sc_agg(og1, og2, src, dst):
    @functools.partial(
        pl.kernel,
        out_type=jax.ShapeDtypeStruct((2, NC, NPAD, DH), jnp.float32),
        mesh=_mesh,
        scratch_types=(
            [pltpu.VMEM((K,), jnp.int32)] * 6
            + [pltpu.VMEM((K, DH), jnp.float32)] * 3
            + [pltpu.VMEM((REM,), jnp.int32)] * 2
            + [pltpu.VMEM((REM, DH), jnp.float32)]
            + [pltpu.VMEM_SHARED((NPAD, DH), jnp.float32)]
            + [pltpu.SemaphoreType.DMA] * 5
        ),
        compiler_params=_sc_params,
    )
    def k(g1_hbm, g2_hbm, src_hbm, dst_hbm, out_hbm,
          src0, src1, src2, dst0, dst1, dst2, hb0, hb1, hb2,
          srcr, dstr, hr, table,
          semi, semg0, semg1, semg2, sems):
        c = lax.axis_index("c")
        s = lax.axis_index("s")
        wid = s * NC + c
        row0 = s * RPS
        ebase = wid * EPW

        SB = (src0, src1, src2)
        DB = (dst0, dst1, dst2)
        HB = (hb0, hb1, hb2)
        SEMG = (semg0, semg1, semg2)

        zeros16 = jnp.zeros((16,), jnp.float32)

        def idx_issue(p, b):
            pltpu.async_copy(src_hbm.at[pl.ds(b, K)], SB[p], semi)
            pltpu.async_copy(dst_hbm.at[pl.ds(b, K)], DB[p], semi)

        def idx_wait(p, b):
            pltpu.make_async_copy(src_hbm.at[pl.ds(b, K)], SB[p],
                                  semi).wait()
            pltpu.make_async_copy(dst_hbm.at[pl.ds(b, K)], DB[p],
                                  semi).wait()

        for phase, g_hbm in ((0, g1_hbm), (1, g2_hbm)):
            def gat_issue(p, _g=g_hbm):
                pltpu.async_copy(_g.at[SB[p]], HB[p], SEMG[p])

            def gat_wait(p, _g=g_hbm):
                pltpu.make_async_copy(_g.at[SB[p]], HB[p], SEMG[p]).wait()

            def scat_issue(p):
                pltpu.async_copy(HB[p], table.at[DB[p]], sems, add=True)

            def scat_wait(p):
                pltpu.make_async_copy(HB[p], table.at[DB[p]],
                                      sems).wait()

            def body(bw, p, prev_scat=True, nxt=True, nxt2=True):
                q = (p + 1) % 3
                r = (p + 2) % 3
                if nxt:
                    idx_wait(q, bw + K)
                    gat_issue(q)
                if prev_scat:
                    scat_wait(r)
                if nxt2:
                    idx_issue(r, bw + 2 * K)
                gat_wait(p)
                scat_issue(p)

            @pl.loop(0, K)
            def _(kk):
                for j in range(DH // 16):
                    hb0[kk, pl.ds(j * 16, 16)] = zeros16

            @pl.loop(0, RPS // K)
            def _(i):
                pltpu.sync_copy(hb0, table.at[pl.ds(row0 + i * K, K)])

            plsc.subcore_barrier()

            idx_issue(0, ebase)
            idx_wait(0, ebase)
            gat_issue(0)
            idx_issue(1, ebase + K)
            body(ebase, 0, prev_scat=False)

            @pl.loop(0, (NWF - 3) // 3)
            def _(t):
                b0 = ebase + (3 * t + 1) * K
                body(b0, 1)
                body(b0 + K, 2)
                body(b0 + 2 * K, 0)

            bw0 = ebase + (NWF - 2) * K
            body(bw0, 1, nxt2=False)
            body(bw0 + K, 2, nxt=False, nxt2=False)
            scat_wait(2)

            er = ebase + NWF * K
            pltpu.sync_copy(src_hbm.at[pl.ds(er, REM)], srcr)
            pltpu.sync_copy(dst_hbm.at[pl.ds(er, REM)], dstr)
            pltpu.sync_copy(g_hbm.at[srcr], hr)
            pltpu.sync_copy(hr, table.at[dstr], add=True)

            plsc.subcore_barrier()
            pltpu.sync_copy(table.at[pl.ds(row0, RPS)],
                            out_hbm.at[phase, c, pl.ds(row0, RPS)])

    return k(og1, og2, src, dst)


def _tc_post(agg_p, og1, og2, deg, W_l, b_l, W_r):
    def body(a_ref, g1_ref, g2_ref, d_ref, wl_ref, bl_ref, wr_ref, o_ref):
        a1 = a_ref[0, 0] + a_ref[0, 1]
        a2 = a_ref[1, 0] + a_ref[1, 1]
        a = (jnp.concatenate([a1[:N], a2[:N]], axis=1)
             / jnp.maximum(d_ref[...], 1.0))
        og = jnp.concatenate([g1_ref[...], g2_ref[...]], axis=1)
        out = (jnp.dot(a, wl_ref[...], preferred_element_type=jnp.float32)
               + jnp.dot(og, wr_ref[...],
                         preferred_element_type=jnp.float32)
               + bl_ref[...][None, :])
        nrm = jnp.sqrt(jnp.sum(out * out, axis=1, keepdims=True))
        o_ref[...] = out / jnp.maximum(nrm, 1e-12)

    return pl.pallas_call(
        body,
        out_shape=jax.ShapeDtypeStruct((N, D), jnp.float32),
    )(agg_p, og1, og2, deg, W_l, b_l, W_r)


def kernel(x, edge_index, W_gat, att_src, att_dst, b_gat, W_l, b_l, W_r):
    src = edge_index[0]
    dst = edge_index[1]
    h1, h2, asrc2, adst2, cvec = _tc_pre(
        x, W_gat, att_src.reshape(D, 1), att_dst.reshape(D, 1))
    asrc = asrc2.reshape(N)
    adst = adst2.reshape(N)
    gat_p, den_p, deg_p = _sc_gat(h1, h2, asrc, adst, src, dst, cvec)
    og1, og2, deg = _tc_mid(gat_p, den_p.reshape(NW, NPAD),
                            deg_p.reshape(NW, NPAD), b_gat)
    agg_p = _sc_agg(og1, og2, src, dst)
    return _tc_post(agg_p, og1, og2, deg, W_l, b_l, W_r)
